# speedup vs baseline: 1.0901x; 1.0901x over previous
import math

import jax
import jax.numpy as jnp
from jax.experimental import pallas as pl

_RMAX = 5.0
_G = 64
_CE = 1280


def _basis_body(ps_ref, pd_ref, fr_ref, o_ref):
    dd = ps_ref[...] - pd_ref[...]
    d2 = jnp.sum(dd * dd, axis=1, keepdims=True)
    dist = jnp.sqrt(d2 + 1e-12)
    b = jnp.sin(fr_ref[...] * dist) / jnp.maximum(dist, 1e-6)
    r = dist * (1.0 / _RMAX)
    cut = 1.0 - 6.0 * r**5 + 15.0 * r**4 - 10.0 * r**3
    o_ref[...] = b * cut


def _edge_body(basis_ref, xs_ref, w1_ref, b1_ref, w2_ref, b2_ref, w3_ref,
               b3_ref, o_ref):
    h = jax.nn.silu(basis_ref[...] @ w1_ref[...] + b1_ref[...])
    h = jax.nn.silu(h @ w2_ref[...] + b2_ref[...])
    tpw = h @ w3_ref[...] + b3_ref[...]
    ce = basis_ref.shape[0]
    d = xs_ref.shape[1]
    tpw = tpw.reshape(ce, d, d)
    msg = jnp.sum(xs_ref[...][:, :, None] * tpw, axis=1)
    o_ref[...] = msg * (1.0 / math.sqrt(d))


def _node_body(p_ref, x_ref, wsi_ref, bnw_ref, bnb_ref, o_ref):
    d = x_ref.shape[1]
    agg = jnp.sum(p_ref[...], axis=0)
    agg = jax.nn.silu(agg)
    out = (agg @ wsi_ref[...]) * (1.0 / math.sqrt(d))
    mu = jnp.mean(out, axis=0, keepdims=True)
    c = out - mu
    var = jnp.mean(c * c, axis=0, keepdims=True)
    o_ref[...] = (c / jnp.sqrt(var + 1e-5)) * bnw_ref[...] + bnb_ref[...] + x_ref[...]


def _init_body(sp_ref, emb_ref, win_ref, o_ref):
    n = sp_ref.shape[0]
    nsp = emb_ref.shape[0]
    d = emb_ref.shape[1]
    oh = (sp_ref[...] == jax.lax.broadcasted_iota(jnp.int32, (n, nsp), 1))
    x0 = (oh.astype(jnp.float32) @ emb_ref[...]) @ win_ref[...]
    o_ref[...] = x0 * (1.0 / math.sqrt(d))


def _pool_body(x_ref, b_ref, wh1_ref, bh1_ref, wh2_ref, bh2_ref, o_ref):
    n = x_ref.shape[0]
    oh = (b_ref[...] == jax.lax.broadcasted_iota(jnp.int32, (_G, n), 0))
    oh = oh.astype(jnp.float32)
    sums = oh @ x_ref[...]
    cnts = jnp.sum(oh, axis=1, keepdims=True)
    pooled = sums / jnp.maximum(cnts, 1.0)
    h = pooled @ wh1_ref[...] + bh1_ref[...]
    h = jax.nn.silu(h)
    o_ref[...] = h @ wh2_ref[...] + bh2_ref[...]


def kernel(species, pos, edge_index, batch, embed, freqs, Win, W1, b1, W2, b2,
           W3, b3, Wsi, bnw, bnb, Wh1, bh1, Wh2, bh2):
    N = species.shape[0]
    D = embed.shape[1]
    E = edge_index.shape[1]
    NB = freqs.shape[0]
    L, RH = W1.shape[0], W1.shape[2]
    f32 = jnp.float32
    src, dst = edge_index[0], edge_index[1]

    nsp_pad = 128
    emb_p = jnp.pad(embed, ((0, nsp_pad - embed.shape[0]), (0, 0)))
    x = pl.pallas_call(
        _init_body,
        out_shape=jax.ShapeDtypeStruct((N, D), f32),
    )(species.reshape(N, 1).astype(jnp.int32), emb_p, Win)

    pos4 = jnp.pad(pos, ((0, 0), (0, 1)))
    ps = jnp.take(pos4, src, axis=0)
    pd = jnp.take(pos4, dst, axis=0)
    basis = pl.pallas_call(
        _basis_body,
        grid=(E // _CE,),
        in_specs=[
            pl.BlockSpec((_CE, 4), lambda i: (i, 0)),
            pl.BlockSpec((_CE, 4), lambda i: (i, 0)),
            pl.BlockSpec((1, NB), lambda i: (0, 0)),
        ],
        out_specs=pl.BlockSpec((_CE, NB), lambda i: (i, 0)),
        out_shape=jax.ShapeDtypeStruct((E, NB), f32),
    )(ps, pd, freqs.reshape(1, NB))

    for l in range(L):
        xs = jnp.take(x, src, axis=0)
        msg = pl.pallas_call(
            _edge_body,
            grid=(E // _CE,),
            in_specs=[
                pl.BlockSpec((_CE, NB), lambda i: (i, 0)),
                pl.BlockSpec((_CE, D), lambda i: (i, 0)),
                pl.BlockSpec((NB, RH), lambda i: (0, 0)),
                pl.BlockSpec((1, RH), lambda i: (0, 0)),
                pl.BlockSpec((RH, RH), lambda i: (0, 0)),
                pl.BlockSpec((1, RH), lambda i: (0, 0)),
                pl.BlockSpec((RH, D * D), lambda i: (0, 0)),
                pl.BlockSpec((1, D * D), lambda i: (0, 0)),
            ],
            out_specs=pl.BlockSpec((_CE, D), lambda i: (i, 0)),
            out_shape=jax.ShapeDtypeStruct((E, D), f32),
        )(basis, xs, W1[l], b1[l].reshape(1, RH), W2[l], b2[l].reshape(1, RH),
          W3[l], b3[l].reshape(1, D * D))
        agg = jnp.zeros((N, D), f32).at[dst].add(msg)
        x = pl.pallas_call(
            _node_body,
            out_shape=jax.ShapeDtypeStruct((N, D), f32),
        )(agg[None], x, Wsi[l], bnw[l].reshape(1, D), bnb[l].reshape(1, D))

    out = pl.pallas_call(
        _pool_body,
        out_shape=jax.ShapeDtypeStruct((_G, 1), f32),
    )(x, batch.reshape(1, N).astype(jnp.int32), Wh1, bh1.reshape(1, D),
      Wh2, bh2.reshape(1, 1))
    return out
